# baseline (speedup 1.0000x reference)
import functools

import jax
import jax.numpy as jnp
from jax import lax
from jax.experimental import pallas as pl
from jax.experimental.pallas import tpu as pltpu
from jax.experimental.pallas import tpu_sc as plsc

_N0, _N1, _N2 = 10000, 30000, 20000
_D = 128
_SLOPE = 0.2
_CH = 64
_NS = 16
_ZR = 512


def _mm_body(x_ref, w_ref, o_ref):
    o_ref[...] = jnp.dot(x_ref[...], w_ref[...],
                         preferred_element_type=jnp.float32)


def _mm_dual_body(x_ref, w_ref, o1_ref, o2_ref):
    y = jnp.dot(x_ref[...], w_ref[...], preferred_element_type=jnp.float32)
    o1_ref[...] = y
    c = pl.program_id(1)
    o2_ref[...] = jnp.where(c == 0, y[:, :64], y[:, 64:])


def _mm_dual(x, w, bn=400):
    n, d = x.shape
    nb = n // bn
    y, yf = pl.pallas_call(
        _mm_dual_body,
        grid=(nb, 2),
        in_specs=[pl.BlockSpec((bn, d), lambda i, c: (i, 0)),
                  pl.BlockSpec((d, d), lambda i, c: (0, 0))],
        out_specs=[pl.BlockSpec((bn, d), lambda i, c: (i, 0)),
                   pl.BlockSpec((bn, 64), lambda i, c, nb=nb: (c * nb + i, 0))],
        out_shape=[jax.ShapeDtypeStruct((n, d), jnp.float32),
                   jax.ShapeDtypeStruct((2 * n, 64), jnp.float32)],
    )(x, w)
    return y, yf


def _mm(x, w, bn=400):
    n, d = x.shape
    c = w.shape[1]
    return pl.pallas_call(
        _mm_body,
        grid=(n // bn,),
        in_specs=[pl.BlockSpec((bn, d), lambda i: (i, 0)),
                  pl.BlockSpec((d, c), lambda i: (0, 0))],
        out_specs=pl.BlockSpec((bn, c), lambda i: (i, 0)),
        out_shape=jax.ShapeDtypeStruct((n, c), jnp.float32),
    )(x, w)


def _row_norm(vals, rows, n):
    s = jax.ops.segment_sum(vals, rows, num_segments=n)
    return vals / s[rows]


def _hbs(x, idx, W, a, n):
    msg = _mm(x, W)
    i, j = idx[0], idx[1]
    z = jnp.concatenate([msg[i], msg[j]], axis=1)
    e = jax.nn.leaky_relu(z @ a, _SLOPE)[:, 0]
    att = _row_norm(e, i, n)
    return jax.ops.segment_sum(att[:, None] * msg[j], i, num_segments=n)


def _hbns(x_s, x_t, idx, w_s, w_t, a, n_t, n_s):
    s_msg = _mm(x_s, w_s)
    t_msg = _mm(x_t, w_t)
    ti, sj = idx[0], idx[1]
    e = jax.nn.leaky_relu(jnp.concatenate([s_msg[sj], t_msg[ti]], axis=1) @ a, _SLOPE)[:, 0]
    f = jax.nn.leaky_relu(jnp.concatenate([t_msg[ti], s_msg[sj]], axis=1) @ a, _SLOPE)[:, 0]
    e = _row_norm(e, ti, n_t)
    f = _row_norm(f, sj, n_s)
    msg_on_target = jax.ops.segment_sum(e[:, None] * s_msg[sj], ti, num_segments=n_t)
    msg_on_source = jax.ops.segment_sum(f[:, None] * t_msg[ti], sj, num_segments=n_s)
    return msg_on_source, msg_on_target


def _hbs_att(x, idx, W, a, n):
    msg, msgf = _mm_dual(x, W)
    i, j = idx[0], idx[1]
    z = jnp.concatenate([msg[i], msg[j]], axis=1)
    e = jax.nn.leaky_relu(z @ a, _SLOPE)[:, 0]
    return _row_norm(e, i, n), msgf


def _hbns_att(x_s, x_t, idx, w_s, w_t, a, n_t, n_s):
    s_msg, s_msgf = _mm_dual(x_s, w_s)
    t_msg, t_msgf = _mm_dual(x_t, w_t)
    ti, sj = idx[0], idx[1]
    e = jax.nn.leaky_relu(jnp.concatenate([s_msg[sj], t_msg[ti]], axis=1) @ a, _SLOPE)[:, 0]
    f = jax.nn.leaky_relu(jnp.concatenate([t_msg[ti], s_msg[sj]], axis=1) @ a, _SLOPE)[:, 0]
    e = _row_norm(e, ti, n_t)
    f = _row_norm(f, sj, n_s)
    return e, f, s_msgf, t_msgf


def _pad_edges(rows, cols, att, n_out, n_src):
    nnz = rows.shape[0]
    npad = (-nnz) % (_NS * _CH * 4) + _NS * _CH * 4
    k = jnp.arange(npad, dtype=jnp.int32)
    rows_p = jnp.concatenate([rows, n_out + (k % 16)])
    cols_p = jnp.concatenate([cols, k % 16])
    att_p = jnp.concatenate([att, jnp.zeros((npad,), jnp.float32)])
    return (rows_p.reshape(-1, _CH), cols_p.reshape(-1, _CH),
            att_p.reshape(-1, _CH))


def _sc_group(n_out, agg_specs):
    n_acc = ((n_out + 16 + 127) // 128) * 128
    racc = n_acc // _NS
    mesh = plsc.VectorSubcoreMesh(core_axis_name="c", subcore_axis_name="s")

    def body(*refs):
        ins = refs[:4 * len(agg_specs)]
        out = refs[4 * len(agg_specs)]
        (acc, rows_sup, cols_sup, att_sup, bufa, bufb,
         gsa, gsb, ssa, ssb) = refs[4 * len(agg_specs) + 1:]
        c = lax.axis_index("c")
        s = lax.axis_index("s")

        def zfill(i, _):
            for r in range(4):
                bufa[i, pl.ds(r * 16, 16)] = jnp.zeros((16,), jnp.float32)
            return _
        lax.fori_loop(0, _CH, zfill, None)

        base_r = s * racc
        nfull, rem = divmod(racc, _CH)
        for j in range(nfull):
            pltpu.sync_copy(bufa, acc.at[pl.ds(base_r + j * _CH, _CH)])
        if rem:
            pltpu.sync_copy(bufa.at[pl.ds(0, rem)],
                            acc.at[pl.ds(base_r + nfull * _CH, rem)])
        plsc.subcore_barrier()

        bufs = (bufa, bufb)
        gsems = (gsa, gsb)
        ssems = (ssa, ssb)

        for a_i, (nnz_p, n_src) in enumerate(agg_specs):
            rows_h, cols_h, att_h, msg_h = ins[4 * a_i: 4 * a_i + 4]
            ept = nnz_p // _NS
            nsup = ept // (4 * _CH)
            srow0 = s * (ept // _CH)
            half = c * n_src

            def load_super(sp):
                pltpu.sync_copy(rows_h.at[pl.ds(srow0 + sp * 4, 4)], rows_sup)
                pltpu.sync_copy(cols_h.at[pl.ds(srow0 + sp * 4, 4)], cols_sup)
                pltpu.sync_copy(att_h.at[pl.ds(srow0 + sp * 4, 4)], att_sup)
                for j in range(4):
                    for t in range(_CH // 16):
                        cols_sup[j, pl.ds(t * 16, 16)] = (
                            cols_sup[j, pl.ds(t * 16, 16)]
                            + jnp.full((16,), half, jnp.int32))

            def gather(j, p):
                pltpu.async_copy(msg_h.at[cols_sup.at[j]], bufs[p], gsems[p])

            def gwait(j, p):
                pltpu.make_async_copy(msg_h.at[cols_sup.at[j]], bufs[p],
                                      gsems[p]).wait()

            def scatter(j, p):
                pltpu.async_copy(bufs[p], acc.at[rows_sup.at[j]],
                                 ssems[p], add=True)

            def swait(j, p):
                pltpu.make_async_copy(bufs[p], acc.at[rows_sup.at[j]],
                                      ssems[p]).wait()

            def scale(j, p):
                for g in range(_CH // 16):
                    ev = att_sup[j, pl.ds(g * 16, 16)]
                    for l in range(16):
                        b = jnp.full((16,), ev[l])
                        i = g * 16 + l
                        for r in range(4):
                            bufs[p][i, pl.ds(r * 16, 16)] = (
                                bufs[p][i, pl.ds(r * 16, 16)] * b)

            load_super(0)
            gather(0, 0)

            def super_body(sp, _):
                gwait(0, 0)
                gather(1, 1)
                scale(0, 0)
                scatter(0, 0)
                gwait(1, 1)
                swait(0, 0)
                gather(2, 0)
                scale(1, 1)
                scatter(1, 1)
                gwait(2, 0)
                swait(1, 1)
                gather(3, 1)
                scale(2, 0)
                scatter(2, 0)
                gwait(3, 1)
                scale(3, 1)
                scatter(3, 1)
                swait(2, 0)
                swait(3, 1)
                load_super(sp + 1)
                gather(0, 0)
                return _
            lax.fori_loop(0, nsup, super_body, None)
            gwait(0, 0)

        plsc.subcore_barrier()
        pltpu.sync_copy(acc.at[pl.ds(base_r, racc)],
                        out.at[pl.ds(base_r, racc), pl.ds(c * 64, 64)])

    return pl.kernel(
        body,
        out_type=jax.ShapeDtypeStruct((n_acc, _D), jnp.float32),
        mesh=mesh,
        compiler_params=pltpu.CompilerParams(use_tc_tiling_on_sc=False),
        scratch_types=[
            pltpu.VMEM_SHARED((n_acc, 64), jnp.float32),
            pltpu.VMEM((4, _CH), jnp.int32),
            pltpu.VMEM((4, _CH), jnp.int32),
            pltpu.VMEM((4, _CH), jnp.float32),
            pltpu.VMEM((_CH, 64), jnp.float32),
            pltpu.VMEM((_CH, 64), jnp.float32),
            pltpu.SemaphoreType.DMA,
            pltpu.SemaphoreType.DMA,
            pltpu.SemaphoreType.DMA,
            pltpu.SemaphoreType.DMA,
        ],
    )


def _run_group(n_out, aggs):
    specs = []
    args = []
    for rows, cols, att, msg, n_src in aggs:
        rows_p, cols_p, att_p = _pad_edges(rows, cols, att, n_out, n_src)
        specs.append((rows_p.shape[0] * _CH - _NS * _CH * 4, n_src))
        args += [rows_p, cols_p, att_p, msg]
    out = _sc_group(n_out, tuple(specs))(*args)
    return out[:n_out]


def kernel(x_0, x_1, x_2, adjacency_0, adjacency_1, coadjacency_2,
           incidence_1, incidence_2, params):
    p = params

    x_0_to_0 = _hbs(x_0, adjacency_0, p["hbs_0_l1_w"], p["hbs_0_l1_a"], _N0)
    x_0_to_1, x_1_to_0 = _hbns(x_1, x_0, incidence_1, p["hbns_01_l1_ws"],
                               p["hbns_01_l1_wt"], p["hbns_01_l1_a"], _N0, _N1)
    x_1_to_2, x_2_to_1 = _hbns(x_2, x_1, incidence_2, p["hbns_12_l1_ws"],
                               p["hbns_12_l1_wt"], p["hbns_12_l1_a"], _N1, _N2)
    x_0_l1 = x_0_to_0 + x_1_to_0
    x_1_l1 = x_0_to_1 + x_2_to_1
    x_2_l1 = x_1_to_2

    att_a0, msg0 = _hbs_att(x_0_l1, adjacency_0, p["hbs_0_l2_w"],
                            p["hbs_0_l2_a"], _N0)
    e01, f01, s01, t01 = _hbns_att(x_1_l1, x_0_l1, incidence_1,
                                   p["hbns_01_l2_ws"], p["hbns_01_l2_wt"],
                                   p["hbns_01_l2_a"], _N0, _N1)
    att_a1, msg1 = _hbs_att(x_1_l1, adjacency_1, p["hbs_1_l2_w"],
                            p["hbs_1_l2_a"], _N1)
    e12, f12, s12, t12 = _hbns_att(x_2_l1, x_1_l1, incidence_2,
                                   p["hbns_12_l2_ws"], p["hbns_12_l2_wt"],
                                   p["hbns_12_l2_a"], _N1, _N2)
    att_a2, msg2 = _hbs_att(x_2_l1, coadjacency_2, p["hbs_2_l2_w"],
                            p["hbs_2_l2_a"], _N2)

    r_a0, c_a0 = adjacency_0[0], adjacency_0[1]
    r_i1, c_i1 = incidence_1[0], incidence_1[1]
    r_a1, c_a1 = adjacency_1[0], adjacency_1[1]
    r_i2, c_i2 = incidence_2[0], incidence_2[1]
    r_a2, c_a2 = coadjacency_2[0], coadjacency_2[1]

    x_0_l2 = _run_group(_N0, [
        (r_a0, c_a0, att_a0, msg0, _N0),
        (r_i1, c_i1, e01, s01, _N1),
    ])
    x_1_l2 = _run_group(_N1, [
        (c_i1, r_i1, f01, t01, _N0),
        (r_a1, c_a1, att_a1, msg1, _N1),
        (r_i2, c_i2, e12, s12, _N2),
    ])
    x_2_l2 = _run_group(_N2, [
        (c_i2, r_i2, f12, t12, _N1),
        (r_a2, c_a2, att_a2, msg2, _N2),
    ])
    return x_0_l2, x_1_l2, x_2_l2


# speedup vs baseline: 1.0357x; 1.0357x over previous
import functools

import jax
import jax.numpy as jnp
from jax import lax
from jax.experimental import pallas as pl
from jax.experimental.pallas import tpu as pltpu
from jax.experimental.pallas import tpu_sc as plsc

_N0, _N1, _N2 = 10000, 30000, 20000
_D = 128
_SLOPE = 0.2
_CH = 64
_NS = 16
_ZR = 512


def _mm_body(x_ref, w_ref, o_ref):
    o_ref[...] = jnp.dot(x_ref[...], w_ref[...],
                         preferred_element_type=jnp.float32)


def _mm(x, w, bn=400):
    n, d = x.shape
    c = w.shape[1]
    return pl.pallas_call(
        _mm_body,
        grid=(n // bn,),
        in_specs=[pl.BlockSpec((bn, d), lambda i: (i, 0)),
                  pl.BlockSpec((d, c), lambda i: (0, 0))],
        out_specs=pl.BlockSpec((bn, c), lambda i: (i, 0)),
        out_shape=jax.ShapeDtypeStruct((n, c), jnp.float32),
    )(x, w)


def _row_norm(vals, rows, n):
    s = jax.ops.segment_sum(vals, rows, num_segments=n)
    return vals / s[rows]


def _hbs(x, idx, W, a, n):
    msg = _mm(x, W)
    i, j = idx[0], idx[1]
    z = jnp.concatenate([msg[i], msg[j]], axis=1)
    e = jax.nn.leaky_relu(z @ a, _SLOPE)[:, 0]
    att = _row_norm(e, i, n)
    return jax.ops.segment_sum(att[:, None] * msg[j], i, num_segments=n)


def _hbns(x_s, x_t, idx, w_s, w_t, a, n_t, n_s):
    s_msg = _mm(x_s, w_s)
    t_msg = _mm(x_t, w_t)
    ti, sj = idx[0], idx[1]
    e = jax.nn.leaky_relu(jnp.concatenate([s_msg[sj], t_msg[ti]], axis=1) @ a, _SLOPE)[:, 0]
    f = jax.nn.leaky_relu(jnp.concatenate([t_msg[ti], s_msg[sj]], axis=1) @ a, _SLOPE)[:, 0]
    e = _row_norm(e, ti, n_t)
    f = _row_norm(f, sj, n_s)
    msg_on_target = jax.ops.segment_sum(e[:, None] * s_msg[sj], ti, num_segments=n_t)
    msg_on_source = jax.ops.segment_sum(f[:, None] * t_msg[ti], sj, num_segments=n_s)
    return msg_on_source, msg_on_target


def _hbs_att(x, idx, W, a, n):
    msg = _mm(x, W)
    i, j = idx[0], idx[1]
    z = jnp.concatenate([msg[i], msg[j]], axis=1)
    e = jax.nn.leaky_relu(z @ a, _SLOPE)[:, 0]
    return _row_norm(e, i, n), msg.reshape(-1, 64)


def _hbns_att(x_s, x_t, idx, w_s, w_t, a, n_t, n_s):
    s_msg = _mm(x_s, w_s)
    t_msg = _mm(x_t, w_t)
    ti, sj = idx[0], idx[1]
    e = jax.nn.leaky_relu(jnp.concatenate([s_msg[sj], t_msg[ti]], axis=1) @ a, _SLOPE)[:, 0]
    f = jax.nn.leaky_relu(jnp.concatenate([t_msg[ti], s_msg[sj]], axis=1) @ a, _SLOPE)[:, 0]
    e = _row_norm(e, ti, n_t)
    f = _row_norm(f, sj, n_s)
    return e, f, s_msg.reshape(-1, 64), t_msg.reshape(-1, 64)


def _pad_edges(rows, cols, att, n_out, n_src):
    nnz = rows.shape[0]
    npad = (-nnz) % (_NS * _CH * 4) + _NS * _CH * 4
    k = jnp.arange(npad, dtype=jnp.int32)
    rows_p = jnp.concatenate([rows, n_out + (k % 16)])
    cols_p = jnp.concatenate([cols, k % 16])
    att_p = jnp.concatenate([att, jnp.zeros((npad,), jnp.float32)])
    return (rows_p.reshape(-1, _CH), cols_p.reshape(-1, _CH),
            att_p.reshape(-1, _CH))


def _sc_group(n_out, agg_specs):
    n_acc = ((n_out + 16 + 127) // 128) * 128
    racc = n_acc // _NS
    mesh = plsc.VectorSubcoreMesh(core_axis_name="c", subcore_axis_name="s")

    def body(*refs):
        ins = refs[:4 * len(agg_specs)]
        out = refs[4 * len(agg_specs)]
        (acc, rows_sup, cols_sup, att_sup, bufa, bufb,
         gsa, gsb, ssa, ssb) = refs[4 * len(agg_specs) + 1:]
        c = lax.axis_index("c")
        s = lax.axis_index("s")

        def zfill(i, _):
            for r in range(4):
                bufa[i, pl.ds(r * 16, 16)] = jnp.zeros((16,), jnp.float32)
            return _
        lax.fori_loop(0, _CH, zfill, None)

        base_r = s * racc
        nfull, rem = divmod(racc, _CH)
        for j in range(nfull):
            pltpu.sync_copy(bufa, acc.at[pl.ds(base_r + j * _CH, _CH)])
        if rem:
            pltpu.sync_copy(bufa.at[pl.ds(0, rem)],
                            acc.at[pl.ds(base_r + nfull * _CH, rem)])
        plsc.subcore_barrier()

        bufs = (bufa, bufb)
        gsems = (gsa, gsb)
        ssems = (ssa, ssb)

        for a_i, (nnz_p, n_src) in enumerate(agg_specs):
            rows_h, cols_h, att_h, msg_h = ins[4 * a_i: 4 * a_i + 4]
            ept = nnz_p // _NS
            nsup = ept // (4 * _CH)
            srow0 = s * (ept // _CH)

            def load_super(sp):
                pltpu.sync_copy(rows_h.at[pl.ds(srow0 + sp * 4, 4)], rows_sup)
                pltpu.sync_copy(cols_h.at[pl.ds(srow0 + sp * 4, 4)], cols_sup)
                pltpu.sync_copy(att_h.at[pl.ds(srow0 + sp * 4, 4)], att_sup)
                for j in range(4):
                    for t in range(_CH // 16):
                        cols_sup[j, pl.ds(t * 16, 16)] = (
                            cols_sup[j, pl.ds(t * 16, 16)] * 2
                            + jnp.full((16,), c, jnp.int32))

            def gather(j, p):
                pltpu.async_copy(msg_h.at[cols_sup.at[j]], bufs[p], gsems[p])

            def gwait(j, p):
                pltpu.make_async_copy(msg_h.at[cols_sup.at[j]], bufs[p],
                                      gsems[p]).wait()

            def scatter(j, p):
                pltpu.async_copy(bufs[p], acc.at[rows_sup.at[j]],
                                 ssems[p], add=True)

            def swait(j, p):
                pltpu.make_async_copy(bufs[p], acc.at[rows_sup.at[j]],
                                      ssems[p]).wait()

            def scale(j, p):
                for g in range(_CH // 16):
                    ev = att_sup[j, pl.ds(g * 16, 16)]
                    for l in range(16):
                        b = jnp.full((16,), ev[l])
                        i = g * 16 + l
                        for r in range(4):
                            bufs[p][i, pl.ds(r * 16, 16)] = (
                                bufs[p][i, pl.ds(r * 16, 16)] * b)

            load_super(0)
            gather(0, 0)

            def super_body(sp, _):
                gwait(0, 0)
                gather(1, 1)
                scale(0, 0)
                scatter(0, 0)
                gwait(1, 1)
                swait(0, 0)
                gather(2, 0)
                scale(1, 1)
                scatter(1, 1)
                gwait(2, 0)
                swait(1, 1)
                gather(3, 1)
                scale(2, 0)
                scatter(2, 0)
                gwait(3, 1)
                scale(3, 1)
                scatter(3, 1)
                swait(2, 0)
                swait(3, 1)
                load_super(sp + 1)
                gather(0, 0)
                return _
            lax.fori_loop(0, nsup, super_body, None)
            gwait(0, 0)

        plsc.subcore_barrier()
        pltpu.sync_copy(acc.at[pl.ds(base_r, racc)],
                        out.at[pl.ds(base_r, racc), pl.ds(c * 64, 64)])

    return pl.kernel(
        body,
        out_type=jax.ShapeDtypeStruct((n_acc, _D), jnp.float32),
        mesh=mesh,
        compiler_params=pltpu.CompilerParams(use_tc_tiling_on_sc=False),
        scratch_types=[
            pltpu.VMEM_SHARED((n_acc, 64), jnp.float32),
            pltpu.VMEM((4, _CH), jnp.int32),
            pltpu.VMEM((4, _CH), jnp.int32),
            pltpu.VMEM((4, _CH), jnp.float32),
            pltpu.VMEM((_CH, 64), jnp.float32),
            pltpu.VMEM((_CH, 64), jnp.float32),
            pltpu.SemaphoreType.DMA,
            pltpu.SemaphoreType.DMA,
            pltpu.SemaphoreType.DMA,
            pltpu.SemaphoreType.DMA,
        ],
    )


def _run_group(n_out, aggs):
    specs = []
    args = []
    for rows, cols, att, msg, n_src in aggs:
        rows_p, cols_p, att_p = _pad_edges(rows, cols, att, n_out, n_src)
        specs.append((rows_p.shape[0] * _CH - _NS * _CH * 4, n_src))
        args += [rows_p, cols_p, att_p, msg]
    out = _sc_group(n_out, tuple(specs))(*args)
    return out[:n_out]


def kernel(x_0, x_1, x_2, adjacency_0, adjacency_1, coadjacency_2,
           incidence_1, incidence_2, params):
    p = params

    x_0_to_0 = _hbs(x_0, adjacency_0, p["hbs_0_l1_w"], p["hbs_0_l1_a"], _N0)
    x_0_to_1, x_1_to_0 = _hbns(x_1, x_0, incidence_1, p["hbns_01_l1_ws"],
                               p["hbns_01_l1_wt"], p["hbns_01_l1_a"], _N0, _N1)
    x_1_to_2, x_2_to_1 = _hbns(x_2, x_1, incidence_2, p["hbns_12_l1_ws"],
                               p["hbns_12_l1_wt"], p["hbns_12_l1_a"], _N1, _N2)
    x_0_l1 = x_0_to_0 + x_1_to_0
    x_1_l1 = x_0_to_1 + x_2_to_1
    x_2_l1 = x_1_to_2

    att_a0, msg0 = _hbs_att(x_0_l1, adjacency_0, p["hbs_0_l2_w"],
                            p["hbs_0_l2_a"], _N0)
    e01, f01, s01, t01 = _hbns_att(x_1_l1, x_0_l1, incidence_1,
                                   p["hbns_01_l2_ws"], p["hbns_01_l2_wt"],
                                   p["hbns_01_l2_a"], _N0, _N1)
    att_a1, msg1 = _hbs_att(x_1_l1, adjacency_1, p["hbs_1_l2_w"],
                            p["hbs_1_l2_a"], _N1)
    e12, f12, s12, t12 = _hbns_att(x_2_l1, x_1_l1, incidence_2,
                                   p["hbns_12_l2_ws"], p["hbns_12_l2_wt"],
                                   p["hbns_12_l2_a"], _N1, _N2)
    att_a2, msg2 = _hbs_att(x_2_l1, coadjacency_2, p["hbs_2_l2_w"],
                            p["hbs_2_l2_a"], _N2)

    r_a0, c_a0 = adjacency_0[0], adjacency_0[1]
    r_i1, c_i1 = incidence_1[0], incidence_1[1]
    r_a1, c_a1 = adjacency_1[0], adjacency_1[1]
    r_i2, c_i2 = incidence_2[0], incidence_2[1]
    r_a2, c_a2 = coadjacency_2[0], coadjacency_2[1]

    x_0_l2 = _run_group(_N0, [
        (r_a0, c_a0, att_a0, msg0, _N0),
        (r_i1, c_i1, e01, s01, _N1),
    ])
    x_1_l2 = _run_group(_N1, [
        (c_i1, r_i1, f01, t01, _N0),
        (r_a1, c_a1, att_a1, msg1, _N1),
        (r_i2, c_i2, e12, s12, _N2),
    ])
    x_2_l2 = _run_group(_N2, [
        (c_i2, r_i2, f12, t12, _N1),
        (r_a2, c_a2, att_a2, msg2, _N2),
    ])
    return x_0_l2, x_1_l2, x_2_l2


# speedup vs baseline: 1.0370x; 1.0013x over previous
import jax
import jax.numpy as jnp
from jax import lax
from jax.experimental import pallas as pl
from jax.experimental.pallas import tpu as pltpu
from jax.experimental.pallas import tpu_sc as plsc

_N0, _N1, _N2 = 10000, 30000, 20000
_D = 128
_SLOPE = 0.2
_CH = 64
_NS = 16


def _mm_body(x_ref, w_ref, o_ref):
    o_ref[...] = jnp.dot(x_ref[...], w_ref[...],
                         preferred_element_type=jnp.float32)


def _mm(x, w, bn=400):
    n, d = x.shape
    c = w.shape[1]
    return pl.pallas_call(
        _mm_body,
        grid=(n // bn,),
        in_specs=[pl.BlockSpec((bn, d), lambda i: (i, 0)),
                  pl.BlockSpec((d, c), lambda i: (0, 0))],
        out_specs=pl.BlockSpec((bn, c), lambda i: (i, 0)),
        out_shape=jax.ShapeDtypeStruct((n, c), jnp.float32),
    )(x, w)


def _row_norm(vals, rows, n):
    s = jax.ops.segment_sum(vals, rows, num_segments=n)
    return vals / s[rows]


def _hbs(x, idx, W, a, n):
    msg = _mm(x, W)
    i, j = idx[0], idx[1]
    z = jnp.concatenate([msg[i], msg[j]], axis=1)
    e = jax.nn.leaky_relu(z @ a, _SLOPE)[:, 0]
    att = _row_norm(e, i, n)
    return jax.ops.segment_sum(att[:, None] * msg[j], i, num_segments=n)


def _hbns(x_s, x_t, idx, w_s, w_t, a, n_t, n_s):
    s_msg = _mm(x_s, w_s)
    t_msg = _mm(x_t, w_t)
    ti, sj = idx[0], idx[1]
    e = jax.nn.leaky_relu(jnp.concatenate([s_msg[sj], t_msg[ti]], axis=1) @ a, _SLOPE)[:, 0]
    f = jax.nn.leaky_relu(jnp.concatenate([t_msg[ti], s_msg[sj]], axis=1) @ a, _SLOPE)[:, 0]
    e = _row_norm(e, ti, n_t)
    f = _row_norm(f, sj, n_s)
    msg_on_target = jax.ops.segment_sum(e[:, None] * s_msg[sj], ti, num_segments=n_t)
    msg_on_source = jax.ops.segment_sum(f[:, None] * t_msg[ti], sj, num_segments=n_s)
    return msg_on_source, msg_on_target


def _hbs_att(x, idx, W, a, n):
    msg = _mm(x, W)
    i, j = idx[0], idx[1]
    z = jnp.concatenate([msg[i], msg[j]], axis=1)
    e = jax.nn.leaky_relu(z @ a, _SLOPE)[:, 0]
    return _row_norm(e, i, n), msg.reshape(-1, 64)


def _hbns_att(x_s, x_t, idx, w_s, w_t, a, n_t, n_s):
    s_msg = _mm(x_s, w_s)
    t_msg = _mm(x_t, w_t)
    ti, sj = idx[0], idx[1]
    e = jax.nn.leaky_relu(jnp.concatenate([s_msg[sj], t_msg[ti]], axis=1) @ a, _SLOPE)[:, 0]
    f = jax.nn.leaky_relu(jnp.concatenate([t_msg[ti], s_msg[sj]], axis=1) @ a, _SLOPE)[:, 0]
    e = _row_norm(e, ti, n_t)
    f = _row_norm(f, sj, n_s)
    return e, f, s_msg.reshape(-1, 64), t_msg.reshape(-1, 64)


def _pad_edges(rows, cols, att, n_out, n_src):
    nnz = rows.shape[0]
    npad = (-nnz) % (_NS * _CH * 4) + _NS * _CH * 4
    k = jnp.arange(npad, dtype=jnp.int32)
    rows_p = jnp.concatenate([rows, n_out + (k % 16)])
    cols_p = jnp.concatenate([cols, k % 16])
    att_p = jnp.concatenate([att, jnp.zeros((npad,), jnp.float32)])
    return (rows_p.reshape(-1, _CH), cols_p.reshape(-1, _CH),
            att_p.reshape(-1, _CH))


def _sc_group(n_out, agg_specs):
    n_acc = ((n_out + 16 + 127) // 128) * 128
    racc = n_acc // _NS
    mesh = plsc.VectorSubcoreMesh(core_axis_name="c", subcore_axis_name="s")

    def body(*refs):
        ins = refs[:4 * len(agg_specs)]
        out = refs[4 * len(agg_specs)]
        (acc, rows_sup, cols_sup, att_sup, bufa, bufb,
         gsa, gsb, ssa, ssb) = refs[4 * len(agg_specs) + 1:]
        c = lax.axis_index("c")
        s = lax.axis_index("s")

        def zfill(i, _):
            for r in range(4):
                bufa[i, pl.ds(r * 16, 16)] = jnp.zeros((16,), jnp.float32)
            return _
        lax.fori_loop(0, _CH, zfill, None)

        base_r = s * racc
        nfull, rem = divmod(racc, _CH)
        for j in range(nfull):
            pltpu.sync_copy(bufa, acc.at[pl.ds(base_r + j * _CH, _CH)])
        if rem:
            pltpu.sync_copy(bufa.at[pl.ds(0, rem)],
                            acc.at[pl.ds(base_r + nfull * _CH, rem)])
        plsc.subcore_barrier()

        bufs = (bufa, bufb)
        gsems = (gsa, gsb)
        ssems = (ssa, ssb)

        for a_i, (nnz_p, n_src) in enumerate(agg_specs):
            rows_h, cols_h, att_h, msg_h = ins[4 * a_i: 4 * a_i + 4]
            ept = nnz_p // _NS
            nsup = ept // (4 * _CH)
            srow0 = s * (ept // _CH)

            def load_super(sp):
                pltpu.sync_copy(rows_h.at[pl.ds(srow0 + sp * 4, 4)], rows_sup)
                pltpu.sync_copy(cols_h.at[pl.ds(srow0 + sp * 4, 4)], cols_sup)
                pltpu.sync_copy(att_h.at[pl.ds(srow0 + sp * 4, 4)], att_sup)
                for j in range(4):
                    for t in range(_CH // 16):
                        cols_sup[j, pl.ds(t * 16, 16)] = (
                            cols_sup[j, pl.ds(t * 16, 16)] * 2
                            + jnp.full((16,), c, jnp.int32))

            def gather(j, p):
                pltpu.async_copy(msg_h.at[cols_sup.at[j]], bufs[p], gsems[p])

            def gwait(j, p):
                pltpu.make_async_copy(msg_h.at[cols_sup.at[j]], bufs[p],
                                      gsems[p]).wait()

            def scatter(j, p):
                pltpu.async_copy(bufs[p], acc.at[rows_sup.at[j]],
                                 ssems[p], add=True)

            def swait(j, p):
                pltpu.make_async_copy(bufs[p], acc.at[rows_sup.at[j]],
                                      ssems[p]).wait()

            def scale(j, p):
                for g in range(_CH // 16):
                    ev = att_sup[j, pl.ds(g * 16, 16)]
                    for l in range(16):
                        b = jnp.full((16,), ev[l])
                        i = g * 16 + l
                        for r in range(4):
                            bufs[p][i, pl.ds(r * 16, 16)] = (
                                bufs[p][i, pl.ds(r * 16, 16)] * b)

            load_super(0)
            gather(0, 0)

            def super_body(sp, _):
                gwait(0, 0)
                gather(1, 1)
                scale(0, 0)
                scatter(0, 0)
                gwait(1, 1)
                swait(0, 0)
                gather(2, 0)
                scale(1, 1)
                scatter(1, 1)
                gwait(2, 0)
                swait(1, 1)
                gather(3, 1)
                scale(2, 0)
                scatter(2, 0)
                gwait(3, 1)
                scale(3, 1)
                scatter(3, 1)
                swait(2, 0)
                swait(3, 1)
                load_super(sp + 1)
                gather(0, 0)
                return _
            lax.fori_loop(0, nsup, super_body, None)
            gwait(0, 0)

        plsc.subcore_barrier()
        rout = n_out // _NS
        pltpu.sync_copy(acc.at[pl.ds(s * rout, rout)],
                        out.at[pl.ds(s * rout, rout), pl.ds(c * 64, 64)])

    return pl.kernel(
        body,
        out_type=jax.ShapeDtypeStruct((n_out, _D), jnp.float32),
        mesh=mesh,
        compiler_params=pltpu.CompilerParams(use_tc_tiling_on_sc=False),
        scratch_types=[
            pltpu.VMEM_SHARED((n_acc, 64), jnp.float32),
            pltpu.VMEM((4, _CH), jnp.int32),
            pltpu.VMEM((4, _CH), jnp.int32),
            pltpu.VMEM((4, _CH), jnp.float32),
            pltpu.VMEM((_CH, 64), jnp.float32),
            pltpu.VMEM((_CH, 64), jnp.float32),
            pltpu.SemaphoreType.DMA,
            pltpu.SemaphoreType.DMA,
            pltpu.SemaphoreType.DMA,
            pltpu.SemaphoreType.DMA,
        ],
    )


def _run_group(n_out, aggs):
    specs = []
    args = []
    for rows, cols, att, msg, n_src in aggs:
        rows_p, cols_p, att_p = _pad_edges(rows, cols, att, n_out, n_src)
        specs.append((rows_p.shape[0] * _CH - _NS * _CH * 4, n_src))
        args += [rows_p, cols_p, att_p, msg]
    return _sc_group(n_out, tuple(specs))(*args)


def kernel(x_0, x_1, x_2, adjacency_0, adjacency_1, coadjacency_2,
           incidence_1, incidence_2, params):
    p = params

    x_0_to_0 = _hbs(x_0, adjacency_0, p["hbs_0_l1_w"], p["hbs_0_l1_a"], _N0)
    x_0_to_1, x_1_to_0 = _hbns(x_1, x_0, incidence_1, p["hbns_01_l1_ws"],
                               p["hbns_01_l1_wt"], p["hbns_01_l1_a"], _N0, _N1)
    x_1_to_2, x_2_to_1 = _hbns(x_2, x_1, incidence_2, p["hbns_12_l1_ws"],
                               p["hbns_12_l1_wt"], p["hbns_12_l1_a"], _N1, _N2)
    x_0_l1 = x_0_to_0 + x_1_to_0
    x_1_l1 = x_0_to_1 + x_2_to_1
    x_2_l1 = x_1_to_2

    att_a0, msg0 = _hbs_att(x_0_l1, adjacency_0, p["hbs_0_l2_w"],
                            p["hbs_0_l2_a"], _N0)
    e01, f01, s01, t01 = _hbns_att(x_1_l1, x_0_l1, incidence_1,
                                   p["hbns_01_l2_ws"], p["hbns_01_l2_wt"],
                                   p["hbns_01_l2_a"], _N0, _N1)
    att_a1, msg1 = _hbs_att(x_1_l1, adjacency_1, p["hbs_1_l2_w"],
                            p["hbs_1_l2_a"], _N1)
    e12, f12, s12, t12 = _hbns_att(x_2_l1, x_1_l1, incidence_2,
                                   p["hbns_12_l2_ws"], p["hbns_12_l2_wt"],
                                   p["hbns_12_l2_a"], _N1, _N2)
    att_a2, msg2 = _hbs_att(x_2_l1, coadjacency_2, p["hbs_2_l2_w"],
                            p["hbs_2_l2_a"], _N2)

    r_a0, c_a0 = adjacency_0[0], adjacency_0[1]
    r_i1, c_i1 = incidence_1[0], incidence_1[1]
    r_a1, c_a1 = adjacency_1[0], adjacency_1[1]
    r_i2, c_i2 = incidence_2[0], incidence_2[1]
    r_a2, c_a2 = coadjacency_2[0], coadjacency_2[1]

    x_0_l2 = _run_group(_N0, [
        (r_a0, c_a0, att_a0, msg0, _N0),
        (r_i1, c_i1, e01, s01, _N1),
    ])
    x_1_l2 = _run_group(_N1, [
        (c_i1, r_i1, f01, t01, _N0),
        (r_a1, c_a1, att_a1, msg1, _N1),
        (r_i2, c_i2, e12, s12, _N2),
    ])
    x_2_l2 = _run_group(_N2, [
        (c_i2, r_i2, f12, t12, _N1),
        (r_a2, c_a2, att_a2, msg2, _N2),
    ])
    return x_0_l2, x_1_l2, x_2_l2
